# Initial kernel scaffold; baseline (speedup 1.0000x reference)
import functools
import math

import jax
import jax.numpy as jnp
from jax import lax
from jax.experimental import pallas as pl
from jax.experimental.pallas import tpu as pltpu

N = 10000
D = 256
E = 160000
H = 4
C = D // H

ROW_BLK = 400
N_BLKS = N // ROW_BLK


def _proj_body(low_ref, wq, bq, wk, bk, wv, bv, ws, bs, q_o, k_o, v_o, s_o):
    x = low_ref[...]
    q_o[...] = jnp.dot(x, wq[...], preferred_element_type=jnp.float32) + bq[...]
    k_o[...] = jnp.dot(x, wk[...], preferred_element_type=jnp.float32) + bk[...]
    v_o[...] = jnp.dot(x, wv[...], preferred_element_type=jnp.float32) + bv[...]
    s_o[...] = jnp.dot(x, ws[...], preferred_element_type=jnp.float32) + bs[...]


def _proj(low_emb, p):
    row = pl.BlockSpec((ROW_BLK, D), lambda i: (i, 0))
    wsp = pl.BlockSpec((D, D), lambda i: (0, 0))
    bsp = pl.BlockSpec((1, D), lambda i: (0, 0))
    out = jax.ShapeDtypeStruct((N, D), jnp.float32)
    return pl.pallas_call(
        _proj_body,
        grid=(N_BLKS,),
        in_specs=[row, wsp, bsp, wsp, bsp, wsp, bsp, wsp, bsp],
        out_specs=[row, row, row, row],
        out_shape=[out, out, out, out],
    )(low_emb,
      p['tc_wq'], p['tc_bq'].reshape(1, D),
      p['tc_wk'], p['tc_bk'].reshape(1, D),
      p['tc_wv'], p['tc_bv'].reshape(1, D),
      p['tc_wskip'], p['tc_bskip'].reshape(1, D))


def _ln(x, g, b):
    mu = jnp.mean(x, axis=-1, keepdims=True)
    var = jnp.mean((x - mu) ** 2, axis=-1, keepdims=True)
    return (x - mu) * jax.lax.rsqrt(var + 1e-5) * g + b


def _fused_body(high_ref, low_ref, agg_ref, num_ref, d0_ref, d1_ref, skip_ref,
                eps_ref, w1, b1, w2, b2, gln_g, gln_b, tln_g, tln_b,
                h2l_qw, h2l_qb, h2l_kw, h2l_kb, h2l_vw, h2l_vb,
                l2h_qw, l2h_qb, l2h_kw, l2h_kb, l2h_vw, l2h_vb,
                nh_g, nh_b, nl_g, nl_b,
                high_new_ref, low_new_ref):
    high = high_ref[...]
    low = low_ref[...]
    h = (1.0 + eps_ref[0, 0]) * high + agg_ref[...]
    h = jnp.dot(h, w1[...], preferred_element_type=jnp.float32) + b1[...]
    h = 0.5 * h * (1.0 + lax.erf(h * (1.0 / math.sqrt(2.0))))
    h = jnp.dot(h, w2[...], preferred_element_type=jnp.float32) + b2[...]
    high_out = _ln(h, gln_g[...], gln_b[...])
    denom = d0_ref[...] + d1_ref[...]
    r = 1.0 / jnp.maximum(denom[:, :H], 1e-16)
    rfull = jnp.concatenate(
        [jnp.broadcast_to(r[:, hh:hh + 1], (r.shape[0], C)) for hh in range(H)],
        axis=1)
    out = num_ref[...] * rfull + skip_ref[...]
    low_out = _ln(out, tln_g[...], tln_b[...])
    scale = D ** (-0.5)
    q_h2l = jnp.dot(high_out, h2l_qw[...], preferred_element_type=jnp.float32) + h2l_qb[...]
    k_h2l = jnp.dot(low_out, h2l_kw[...], preferred_element_type=jnp.float32) + h2l_kb[...]
    v_h2l = jnp.dot(low_out, h2l_vw[...], preferred_element_type=jnp.float32) + h2l_vb[...]
    a_h2l = jax.nn.sigmoid(jnp.sum(q_h2l * k_h2l, axis=-1, keepdims=True) * scale)
    high_cross = a_h2l * v_h2l
    q_l2h = jnp.dot(low_out, l2h_qw[...], preferred_element_type=jnp.float32) + l2h_qb[...]
    k_l2h = jnp.dot(high_out, l2h_kw[...], preferred_element_type=jnp.float32) + l2h_kb[...]
    v_l2h = jnp.dot(high_out, l2h_vw[...], preferred_element_type=jnp.float32) + l2h_vb[...]
    a_l2h = jax.nn.sigmoid(jnp.sum(q_l2h * k_l2h, axis=-1, keepdims=True) * scale)
    low_cross = a_l2h * v_l2h
    high_new_ref[...] = _ln(high + high_out + high_cross, nh_g[...], nh_b[...])
    low_new_ref[...] = _ln(low + low_out + low_cross, nl_g[...], nl_b[...])


def _fused(high_emb, low_emb, agg, num, d0, d1, skip, p):
    row = pl.BlockSpec((ROW_BLK, D), lambda i: (i, 0))
    row16 = pl.BlockSpec((ROW_BLK, 16), lambda i: (i, 0))
    w1sp = pl.BlockSpec((D, 2 * D), lambda i: (0, 0))
    b1sp = pl.BlockSpec((1, 2 * D), lambda i: (0, 0))
    w2sp = pl.BlockSpec((2 * D, D), lambda i: (0, 0))
    wsp = pl.BlockSpec((D, D), lambda i: (0, 0))
    bsp = pl.BlockSpec((1, D), lambda i: (0, 0))
    ssp = pl.BlockSpec((1, 1), lambda i: (0, 0))
    out = jax.ShapeDtypeStruct((N, D), jnp.float32)
    b = lambda name: p[name].reshape(1, -1)
    return pl.pallas_call(
        _fused_body,
        grid=(N_BLKS,),
        in_specs=[row, row, row, row, row16, row16, row,
                  ssp, w1sp, b1sp, w2sp, bsp, bsp, bsp, bsp, bsp,
                  wsp, bsp, wsp, bsp, wsp, bsp,
                  wsp, bsp, wsp, bsp, wsp, bsp,
                  bsp, bsp, bsp, bsp],
        out_specs=[row, row],
        out_shape=[out, out],
    )(high_emb, low_emb, agg, num, d0, d1, skip,
      p['gin_eps'].reshape(1, 1), p['gin_w1'], b('gin_b1'), p['gin_w2'], b('gin_b2'),
      b('gin_ln_g'), b('gin_ln_b'), b('tc_ln_g'), b('tc_ln_b'),
      p['c_h2l_qw'], b('c_h2l_qb'), p['c_h2l_kw'], b('c_h2l_kb'),
      p['c_h2l_vw'], b('c_h2l_vb'),
      p['c_l2h_qw'], b('c_l2h_qb'), p['c_l2h_kw'], b('c_l2h_kb'),
      p['c_l2h_vw'], b('c_l2h_vb'),
      b('nh_g'), b('nh_b'), b('nl_g'), b('nl_b'))


def kernel(high_emb, low_emb, spatial_edge_index, grn_edge_index, params):
    p = params
    src, dst = spatial_edge_index[0], spatial_edge_index[1]
    gs, gd = grn_edge_index[0], grn_edge_index[1]

    q, k, v, skip = _proj(low_emb, p)

    agg = jnp.zeros_like(high_emb).at[dst].add(high_emb[src])

    qh = q.reshape(N, H, C)
    kh = k.reshape(N, H, C)
    vh = v.reshape(N, H, C)
    alpha = jnp.sum(qh[gd] * kh[gs], axis=-1) / math.sqrt(C)
    ex = jnp.exp(alpha)
    denom = jax.ops.segment_sum(ex, gd, num_segments=N)
    num = jax.ops.segment_sum(ex[:, :, None] * vh[gs], gd, num_segments=N)
    num = num.reshape(N, D)
    d0 = jnp.pad(denom, ((0, 0), (0, 16 - H)))
    d1 = jnp.zeros_like(d0)

    high_new, low_new = _fused(high_emb, low_emb, agg, num, d0, d1, skip, p)
    return high_new, low_new


# speedup vs baseline: 7.8917x; 7.8917x over previous
import functools
import math

import jax
import jax.numpy as jnp
from jax import lax
from jax.experimental import pallas as pl
from jax.experimental.pallas import tpu as pltpu
from jax.experimental.pallas import tpu_sc as plsc

N = 10000
D = 256
E = 160000
H = 4
C = D // H

ROW_BLK = 400
N_BLKS = N // ROW_BLK

NC = 2
NS = 16
DH = D // 2
NP = 10240
RPT = NP // NS
DUMP = 10100
EPAD = 161280
PT_SC = EPAD // NS
BSC = 80

EB = 640
E_BLKS = EPAD // EB

_MESH = plsc.VectorSubcoreMesh(core_axis_name="c", subcore_axis_name="s",
                               num_cores=NC, num_subcores=NS)


def _f32(shape):
    return jax.ShapeDtypeStruct(shape, jnp.float32)


def _gin_agg(h0, h1, src, dst, z128):
    @functools.partial(
        pl.kernel,
        out_type=[_f32((NP, DH)), _f32((NP, DH))],
        mesh=_MESH,
        compiler_params=pltpu.CompilerParams(needs_layout_passes=False),
        scratch_types=[
            pltpu.VMEM_SHARED((NP, DH), jnp.float32),
            pltpu.VMEM((BSC,), jnp.int32),
            pltpu.VMEM((BSC,), jnp.int32),
            pltpu.VMEM((BSC, DH), jnp.float32),
            pltpu.SemaphoreType.DMA,
        ],
    )
    def body(h0_h, h1_h, src_h, dst_h, z_h, o0_h, o1_h,
             acc, idx_v, dst_v, rows_v, sem):
        c = lax.axis_index("c")
        s = lax.axis_index("s")
        pltpu.sync_copy(z_h.at[pl.ds(s * RPT, RPT)], acc.at[pl.ds(s * RPT, RPT)])
        plsc.subcore_barrier()
        for cid, tbl, out in ((0, h0_h, o0_h), (1, h1_h, o1_h)):
            @pl.when(c == cid)
            def _():
                def step(i, carry):
                    base = s * PT_SC + i * BSC
                    pltpu.sync_copy(src_h.at[pl.ds(base, BSC)], idx_v)
                    pltpu.sync_copy(dst_h.at[pl.ds(base, BSC)], dst_v)
                    pltpu.async_copy(tbl.at[idx_v], rows_v, sem).wait()
                    pltpu.sync_copy(rows_v, acc.at[dst_v], add=True)
                    return carry
                lax.fori_loop(0, PT_SC // BSC, step, 0)
                plsc.subcore_barrier()
                pltpu.sync_copy(acc.at[pl.ds(s * RPT, RPT)],
                                out.at[pl.ds(s * RPT, RPT)])

    return body(h0, h1, src, dst, z128)


def _gather3(q0, q1, k0, k1, v0, v1, gd, gs):
    @functools.partial(
        pl.kernel,
        out_type=[_f32((EPAD, DH)) for _ in range(6)],
        mesh=_MESH,
        compiler_params=pltpu.CompilerParams(needs_layout_passes=False),
        scratch_types=[
            pltpu.VMEM((BSC,), jnp.int32),
            pltpu.VMEM((BSC,), jnp.int32),
            pltpu.VMEM((BSC, DH), jnp.float32),
            pltpu.SemaphoreType.DMA,
        ],
    )
    def body(q0_h, q1_h, k0_h, k1_h, v0_h, v1_h, gd_h, gs_h,
             oq0, oq1, ok0, ok1, ov0, ov1,
             gdv, gsv, rows_v, sem):
        c = lax.axis_index("c")
        s = lax.axis_index("s")
        for cid, tq, tk, tv, oq, okk, ov in (
                (0, q0_h, k0_h, v0_h, oq0, ok0, ov0),
                (1, q1_h, k1_h, v1_h, oq1, ok1, ov1)):
            @pl.when(c == cid)
            def _():
                def step(i, carry):
                    base = s * PT_SC + i * BSC
                    pltpu.sync_copy(gd_h.at[pl.ds(base, BSC)], gdv)
                    pltpu.sync_copy(gs_h.at[pl.ds(base, BSC)], gsv)
                    pltpu.async_copy(tq.at[gdv], rows_v, sem).wait()
                    pltpu.sync_copy(rows_v, oq.at[pl.ds(base, BSC)])
                    pltpu.async_copy(tk.at[gsv], rows_v, sem).wait()
                    pltpu.sync_copy(rows_v, okk.at[pl.ds(base, BSC)])
                    pltpu.async_copy(tv.at[gsv], rows_v, sem).wait()
                    pltpu.sync_copy(rows_v, ov.at[pl.ds(base, BSC)])
                    return carry
                lax.fori_loop(0, PT_SC // BSC, step, 0)

    return body(q0, q1, k0, k1, v0, v1, gd, gs)


def _edge_body(qe_ref, ke_ref, ve_ref, m_ref, p_ref, we_ref, exd_ref):
    prod = qe_ref[...] * ke_ref[...]
    s = jnp.dot(prod, m_ref[...], preferred_element_type=jnp.float32)
    exf = jnp.exp(s * (1.0 / math.sqrt(C)))
    we_ref[...] = exf * ve_ref[...]
    exd_ref[...] = jnp.dot(exf, p_ref[...], preferred_element_type=jnp.float32)


def _edge_tc(qe, ke, ve, mmat, pmat):
    row = pl.BlockSpec((EB, D), lambda i: (i, 0))
    msp = pl.BlockSpec((D, D), lambda i: (0, 0))
    out = jax.ShapeDtypeStruct((EPAD, D), jnp.float32)
    return pl.pallas_call(
        _edge_body,
        grid=(E_BLKS,),
        in_specs=[row, row, row, msp, msp],
        out_specs=[row, row],
        out_shape=[out, out],
    )(qe, ke, ve, mmat, pmat)


def _proj_body(low_ref, wq, bq, wk, bk, wv, bv, ws, bs, q_o, k_o, v_o, s_o):
    x = low_ref[...]
    q_o[...] = jnp.dot(x, wq[...], preferred_element_type=jnp.float32) + bq[...]
    k_o[...] = jnp.dot(x, wk[...], preferred_element_type=jnp.float32) + bk[...]
    v_o[...] = jnp.dot(x, wv[...], preferred_element_type=jnp.float32) + bv[...]
    s_o[...] = jnp.dot(x, ws[...], preferred_element_type=jnp.float32) + bs[...]


def _proj(low_emb, p):
    row = pl.BlockSpec((ROW_BLK, D), lambda i: (i, 0))
    wsp = pl.BlockSpec((D, D), lambda i: (0, 0))
    bsp = pl.BlockSpec((1, D), lambda i: (0, 0))
    out = jax.ShapeDtypeStruct((N, D), jnp.float32)
    return pl.pallas_call(
        _proj_body,
        grid=(N_BLKS,),
        in_specs=[row, wsp, bsp, wsp, bsp, wsp, bsp, wsp, bsp],
        out_specs=[row, row, row, row],
        out_shape=[out, out, out, out],
    )(low_emb,
      p['tc_wq'], p['tc_bq'].reshape(1, D),
      p['tc_wk'], p['tc_bk'].reshape(1, D),
      p['tc_wv'], p['tc_bv'].reshape(1, D),
      p['tc_wskip'], p['tc_bskip'].reshape(1, D))


def _ln(x, g, b):
    mu = jnp.mean(x, axis=-1, keepdims=True)
    var = jnp.mean((x - mu) ** 2, axis=-1, keepdims=True)
    return (x - mu) * jax.lax.rsqrt(var + 1e-5) * g + b


def _fused_body(high_ref, low_ref, agg_ref, num_ref, rinv_ref, skip_ref,
                eps_row, w1, b1, w2, b2, gln_g, gln_b, tln_g, tln_b,
                h2l_qw, h2l_qb, h2l_kw, h2l_kb, h2l_vw, h2l_vb,
                l2h_qw, l2h_qb, l2h_kw, l2h_kb, l2h_vw, l2h_vb,
                nh_g, nh_b, nl_g, nl_b,
                high_new_ref, low_new_ref):
    high = high_ref[...]
    low = low_ref[...]
    h = eps_row[...] * high + agg_ref[...]
    h = jnp.dot(h, w1[...], preferred_element_type=jnp.float32) + b1[...]
    h = 0.5 * h * (1.0 + lax.erf(h * (1.0 / math.sqrt(2.0))))
    h = jnp.dot(h, w2[...], preferred_element_type=jnp.float32) + b2[...]
    high_out = _ln(h, gln_g[...], gln_b[...])
    out = num_ref[...] * rinv_ref[...] + skip_ref[...]
    low_out = _ln(out, tln_g[...], tln_b[...])
    scale = D ** (-0.5)
    q_h2l = jnp.dot(high_out, h2l_qw[...], preferred_element_type=jnp.float32) + h2l_qb[...]
    k_h2l = jnp.dot(low_out, h2l_kw[...], preferred_element_type=jnp.float32) + h2l_kb[...]
    v_h2l = jnp.dot(low_out, h2l_vw[...], preferred_element_type=jnp.float32) + h2l_vb[...]
    a_h2l = jax.nn.sigmoid(jnp.sum(q_h2l * k_h2l, axis=-1, keepdims=True) * scale)
    high_cross = a_h2l * v_h2l
    q_l2h = jnp.dot(low_out, l2h_qw[...], preferred_element_type=jnp.float32) + l2h_qb[...]
    k_l2h = jnp.dot(high_out, l2h_kw[...], preferred_element_type=jnp.float32) + l2h_kb[...]
    v_l2h = jnp.dot(high_out, l2h_vw[...], preferred_element_type=jnp.float32) + l2h_vb[...]
    a_l2h = jax.nn.sigmoid(jnp.sum(q_l2h * k_l2h, axis=-1, keepdims=True) * scale)
    low_cross = a_l2h * v_l2h
    high_new_ref[...] = _ln(high + high_out + high_cross, nh_g[...], nh_b[...])
    low_new_ref[...] = _ln(low + low_out + low_cross, nl_g[...], nl_b[...])


def _fused(high_emb, low_emb, agg, num, rinv, skip, p):
    row = pl.BlockSpec((ROW_BLK, D), lambda i: (i, 0))
    w1sp = pl.BlockSpec((D, 2 * D), lambda i: (0, 0))
    b1sp = pl.BlockSpec((1, 2 * D), lambda i: (0, 0))
    w2sp = pl.BlockSpec((2 * D, D), lambda i: (0, 0))
    wsp = pl.BlockSpec((D, D), lambda i: (0, 0))
    bsp = pl.BlockSpec((1, D), lambda i: (0, 0))
    out = jax.ShapeDtypeStruct((N, D), jnp.float32)
    b = lambda name: p[name].reshape(1, -1)
    return pl.pallas_call(
        _fused_body,
        grid=(N_BLKS,),
        in_specs=[row, row, row, row, row, row,
                  bsp, w1sp, b1sp, w2sp, bsp, bsp, bsp, bsp, bsp,
                  wsp, bsp, wsp, bsp, wsp, bsp,
                  wsp, bsp, wsp, bsp, wsp, bsp,
                  bsp, bsp, bsp, bsp],
        out_specs=[row, row],
        out_shape=[out, out],
    )(high_emb, low_emb, agg, num, rinv, skip,
      jnp.broadcast_to(1.0 + p['gin_eps'], (1, D)), p['gin_w1'], b('gin_b1'), p['gin_w2'], b('gin_b2'),
      b('gin_ln_g'), b('gin_ln_b'), b('tc_ln_g'), b('tc_ln_b'),
      p['c_h2l_qw'], b('c_h2l_qb'), p['c_h2l_kw'], b('c_h2l_kb'),
      p['c_h2l_vw'], b('c_h2l_vb'),
      p['c_l2h_qw'], b('c_l2h_qb'), p['c_l2h_kw'], b('c_l2h_kb'),
      p['c_l2h_vw'], b('c_l2h_vb'),
      b('nh_g'), b('nh_b'), b('nl_g'), b('nl_b'))


def kernel(high_emb, low_emb, spatial_edge_index, grn_edge_index, params):
    p = params
    src, dst = spatial_edge_index[0], spatial_edge_index[1]
    gs, gd = grn_edge_index[0], grn_edge_index[1]

    q, k, v, skip = _proj(low_emb, p)

    padn = EPAD - E
    src_p = jnp.pad(src, (0, padn))
    dst_p = jnp.pad(dst, (0, padn), constant_values=DUMP)
    gs_p = jnp.pad(gs, (0, padn))
    gdg_p = jnp.pad(gd, (0, padn))
    gd_p = jnp.pad(gd, (0, padn), constant_values=DUMP)
    z128 = jnp.zeros((NP, DH), jnp.float32)
    iota_e = jnp.arange(EPAD, dtype=jnp.int32)

    a0, a1 = _gin_agg(high_emb[:, :DH], high_emb[:, DH:], src_p, dst_p, z128)
    agg = jnp.concatenate([a0[:N], a1[:N]], axis=1)

    qe0, qe1, ke0, ke1, ve0, ve1 = _gather3(
        q[:, :DH], q[:, DH:], k[:, :DH], k[:, DH:], v[:, :DH], v[:, DH:],
        gdg_p, gs_p)
    qe = jnp.concatenate([qe0, qe1], axis=1)
    ke = jnp.concatenate([ke0, ke1], axis=1)
    ve = jnp.concatenate([ve0, ve1], axis=1)

    lane = jnp.arange(D, dtype=jnp.int32)
    mmat = (lane[:, None] // C == lane[None, :] // C).astype(jnp.float32)
    pmat = ((lane[:, None] % C == 0) &
            (lane[None, :] == lane[:, None] // C)).astype(jnp.float32)

    we, exd = _edge_tc(qe, ke, ve, mmat, pmat)

    n0, n1 = _gin_agg(we[:, :DH], we[:, DH:], iota_e, gd_p, z128)
    num = jnp.concatenate([n0[:N], n1[:N]], axis=1)
    d0, _ = _gin_agg(exd[:, :DH], exd[:, DH:], iota_e, gd_p, z128)
    denom = d0[:N, :H]

    rinv = jnp.repeat(1.0 / jnp.maximum(denom, 1e-16), C, axis=1)
    high_new, low_new = _fused(high_emb, low_emb, agg, num, rinv, skip, p)
    return high_new, low_new


# speedup vs baseline: 9.0670x; 1.1489x over previous
import functools
import math

import jax
import jax.numpy as jnp
from jax import lax
from jax.experimental import pallas as pl
from jax.experimental.pallas import tpu as pltpu
from jax.experimental.pallas import tpu_sc as plsc

N = 10000
D = 256
E = 160000
H = 4
C = D // H

ROW_BLK = 400
N_BLKS = N // ROW_BLK

NC = 2
NS = 16
DH = D // 2
NP = 10240
RPT = NP // NS
DUMP = 10100
EPAD = 161280
PT_SC = EPAD // NS
BSC = 80

EB = 640
E_BLKS = EPAD // EB

_MESH = plsc.VectorSubcoreMesh(core_axis_name="c", subcore_axis_name="s",
                               num_cores=NC, num_subcores=NS)


def _f32(shape):
    return jax.ShapeDtypeStruct(shape, jnp.float32)


def _gin_agg(h0, h1, src, dst, z128):
    @functools.partial(
        pl.kernel,
        out_type=[_f32((NP, DH)), _f32((NP, DH))],
        mesh=_MESH,
        compiler_params=pltpu.CompilerParams(needs_layout_passes=False),
        scratch_types=[
            pltpu.VMEM_SHARED((NP, DH), jnp.float32),
            pltpu.VMEM((BSC,), jnp.int32),
            pltpu.VMEM((BSC,), jnp.int32),
            pltpu.VMEM((BSC, DH), jnp.float32),
            pltpu.SemaphoreType.DMA,
        ],
    )
    def body(h0_h, h1_h, src_h, dst_h, z_h, o0_h, o1_h,
             acc, idx_v, dst_v, rows_v, sem):
        c = lax.axis_index("c")
        s = lax.axis_index("s")
        pltpu.sync_copy(z_h.at[pl.ds(s * RPT, RPT)], acc.at[pl.ds(s * RPT, RPT)])
        plsc.subcore_barrier()
        for cid, tbl, out in ((0, h0_h, o0_h), (1, h1_h, o1_h)):
            @pl.when(c == cid)
            def _():
                def step(i, carry):
                    base = s * PT_SC + i * BSC
                    pltpu.sync_copy(src_h.at[pl.ds(base, BSC)], idx_v)
                    pltpu.sync_copy(dst_h.at[pl.ds(base, BSC)], dst_v)
                    pltpu.async_copy(tbl.at[idx_v], rows_v, sem).wait()
                    pltpu.sync_copy(rows_v, acc.at[dst_v], add=True)
                    return carry
                lax.fori_loop(0, PT_SC // BSC, step, 0)
                plsc.subcore_barrier()
                pltpu.sync_copy(acc.at[pl.ds(s * RPT, RPT)],
                                out.at[pl.ds(s * RPT, RPT)])

    return body(h0, h1, src, dst, z128)


def _gather3(q0, q1, k0, k1, v0, v1, gd, gs):
    @functools.partial(
        pl.kernel,
        out_type=[_f32((EPAD, DH)) for _ in range(6)],
        mesh=_MESH,
        compiler_params=pltpu.CompilerParams(needs_layout_passes=False),
        scratch_types=[
            pltpu.VMEM((BSC,), jnp.int32),
            pltpu.VMEM((BSC,), jnp.int32),
            pltpu.VMEM((BSC, DH), jnp.float32),
            pltpu.VMEM((BSC, DH), jnp.float32),
            pltpu.VMEM((BSC, DH), jnp.float32),
            pltpu.SemaphoreType.DMA,
            pltpu.SemaphoreType.DMA,
            pltpu.SemaphoreType.DMA,
        ],
    )
    def body(q0_h, q1_h, k0_h, k1_h, v0_h, v1_h, gd_h, gs_h,
             oq0, oq1, ok0, ok1, ov0, ov1,
             gdv, gsv, rq, rk, rv, semq, semk, semv):
        c = lax.axis_index("c")
        s = lax.axis_index("s")
        for cid, tq, tk, tv, oq, okk, ov in (
                (0, q0_h, k0_h, v0_h, oq0, ok0, ov0),
                (1, q1_h, k1_h, v1_h, oq1, ok1, ov1)):
            @pl.when(c == cid)
            def _():
                def step(i, carry):
                    base = s * PT_SC + i * BSC
                    pltpu.sync_copy(gd_h.at[pl.ds(base, BSC)], gdv)
                    pltpu.sync_copy(gs_h.at[pl.ds(base, BSC)], gsv)
                    cq = pltpu.async_copy(tq.at[gdv], rq, semq)
                    ck = pltpu.async_copy(tk.at[gsv], rk, semk)
                    cv = pltpu.async_copy(tv.at[gsv], rv, semv)
                    cq.wait()
                    pltpu.sync_copy(rq, oq.at[pl.ds(base, BSC)])
                    ck.wait()
                    pltpu.sync_copy(rk, okk.at[pl.ds(base, BSC)])
                    cv.wait()
                    pltpu.sync_copy(rv, ov.at[pl.ds(base, BSC)])
                    return carry
                lax.fori_loop(0, PT_SC // BSC, step, 0)

    return body(q0, q1, k0, k1, v0, v1, gd, gs)


PT_DN = EPAD // (NC * NS)


def _den_agg(exd16, dst, z16):
    @functools.partial(
        pl.kernel,
        out_type=[_f32((NP, 16)), _f32((NP, 16))],
        mesh=_MESH,
        compiler_params=pltpu.CompilerParams(needs_layout_passes=False),
        scratch_types=[
            pltpu.VMEM_SHARED((NP, 16), jnp.float32),
            pltpu.VMEM((BSC,), jnp.int32),
            pltpu.VMEM((BSC, 16), jnp.float32),
        ],
    )
    def body(exd_h, dst_h, z_h, d0_h, d1_h, acc, dstv, rows):
        c = lax.axis_index("c")
        s = lax.axis_index("s")
        pltpu.sync_copy(z_h.at[pl.ds(s * RPT, RPT)], acc.at[pl.ds(s * RPT, RPT)])
        plsc.subcore_barrier()

        def step(i, carry):
            base = c * (NS * PT_DN) + s * PT_DN + i * BSC
            pltpu.sync_copy(dst_h.at[pl.ds(base, BSC)], dstv)
            pltpu.sync_copy(exd_h.at[pl.ds(base, BSC)], rows)
            pltpu.sync_copy(rows, acc.at[dstv], add=True)
            return carry

        lax.fori_loop(0, PT_DN // BSC, step, 0)
        plsc.subcore_barrier()
        for cid, out in ((0, d0_h), (1, d1_h)):
            @pl.when(c == cid)
            def _():
                pltpu.sync_copy(acc.at[pl.ds(s * RPT, RPT)],
                                out.at[pl.ds(s * RPT, RPT)])

    return body(exd16, dst, z16)


def _edge_body(qe_ref, ke_ref, ve_ref, m_ref, p_ref, we_ref, exd_ref):
    prod = qe_ref[...] * ke_ref[...]
    s = jnp.dot(prod, m_ref[...], preferred_element_type=jnp.float32)
    exf = jnp.exp(s * (1.0 / math.sqrt(C)))
    we_ref[...] = exf * ve_ref[...]
    exd_ref[...] = jnp.dot(exf, p_ref[...], preferred_element_type=jnp.float32)


def _edge_tc(qe, ke, ve, mmat, pmat):
    row = pl.BlockSpec((EB, D), lambda i: (i, 0))
    msp = pl.BlockSpec((D, D), lambda i: (0, 0))
    out = jax.ShapeDtypeStruct((EPAD, D), jnp.float32)
    return pl.pallas_call(
        _edge_body,
        grid=(E_BLKS,),
        in_specs=[row, row, row, msp, msp],
        out_specs=[row, row],
        out_shape=[out, out],
    )(qe, ke, ve, mmat, pmat)


def _proj_body(low_ref, wq, bq, wk, bk, wv, bv, ws, bs, q_o, k_o, v_o, s_o):
    x = low_ref[...]
    q_o[...] = jnp.dot(x, wq[...], preferred_element_type=jnp.float32) + bq[...]
    k_o[...] = jnp.dot(x, wk[...], preferred_element_type=jnp.float32) + bk[...]
    v_o[...] = jnp.dot(x, wv[...], preferred_element_type=jnp.float32) + bv[...]
    s_o[...] = jnp.dot(x, ws[...], preferred_element_type=jnp.float32) + bs[...]


def _proj(low_emb, p):
    row = pl.BlockSpec((ROW_BLK, D), lambda i: (i, 0))
    wsp = pl.BlockSpec((D, D), lambda i: (0, 0))
    bsp = pl.BlockSpec((1, D), lambda i: (0, 0))
    out = jax.ShapeDtypeStruct((N, D), jnp.float32)
    return pl.pallas_call(
        _proj_body,
        grid=(N_BLKS,),
        in_specs=[row, wsp, bsp, wsp, bsp, wsp, bsp, wsp, bsp],
        out_specs=[row, row, row, row],
        out_shape=[out, out, out, out],
    )(low_emb,
      p['tc_wq'], p['tc_bq'].reshape(1, D),
      p['tc_wk'], p['tc_bk'].reshape(1, D),
      p['tc_wv'], p['tc_bv'].reshape(1, D),
      p['tc_wskip'], p['tc_bskip'].reshape(1, D))


def _ln(x, g, b):
    mu = jnp.mean(x, axis=-1, keepdims=True)
    var = jnp.mean((x - mu) ** 2, axis=-1, keepdims=True)
    return (x - mu) * jax.lax.rsqrt(var + 1e-5) * g + b


def _fused_body(high_ref, low_ref, agg_ref, num_ref, rinv_ref, skip_ref,
                eps_row, w1, b1, w2, b2, gln_g, gln_b, tln_g, tln_b,
                h2l_qw, h2l_qb, h2l_kw, h2l_kb, h2l_vw, h2l_vb,
                l2h_qw, l2h_qb, l2h_kw, l2h_kb, l2h_vw, l2h_vb,
                nh_g, nh_b, nl_g, nl_b,
                high_new_ref, low_new_ref):
    high = high_ref[...]
    low = low_ref[...]
    h = eps_row[...] * high + agg_ref[...]
    h = jnp.dot(h, w1[...], preferred_element_type=jnp.float32) + b1[...]
    h = 0.5 * h * (1.0 + lax.erf(h * (1.0 / math.sqrt(2.0))))
    h = jnp.dot(h, w2[...], preferred_element_type=jnp.float32) + b2[...]
    high_out = _ln(h, gln_g[...], gln_b[...])
    out = num_ref[...] * rinv_ref[...] + skip_ref[...]
    low_out = _ln(out, tln_g[...], tln_b[...])
    scale = D ** (-0.5)
    q_h2l = jnp.dot(high_out, h2l_qw[...], preferred_element_type=jnp.float32) + h2l_qb[...]
    k_h2l = jnp.dot(low_out, h2l_kw[...], preferred_element_type=jnp.float32) + h2l_kb[...]
    v_h2l = jnp.dot(low_out, h2l_vw[...], preferred_element_type=jnp.float32) + h2l_vb[...]
    a_h2l = jax.nn.sigmoid(jnp.sum(q_h2l * k_h2l, axis=-1, keepdims=True) * scale)
    high_cross = a_h2l * v_h2l
    q_l2h = jnp.dot(low_out, l2h_qw[...], preferred_element_type=jnp.float32) + l2h_qb[...]
    k_l2h = jnp.dot(high_out, l2h_kw[...], preferred_element_type=jnp.float32) + l2h_kb[...]
    v_l2h = jnp.dot(high_out, l2h_vw[...], preferred_element_type=jnp.float32) + l2h_vb[...]
    a_l2h = jax.nn.sigmoid(jnp.sum(q_l2h * k_l2h, axis=-1, keepdims=True) * scale)
    low_cross = a_l2h * v_l2h
    high_new_ref[...] = _ln(high + high_out + high_cross, nh_g[...], nh_b[...])
    low_new_ref[...] = _ln(low + low_out + low_cross, nl_g[...], nl_b[...])


def _fused(high_emb, low_emb, agg, num, rinv, skip, p):
    row = pl.BlockSpec((ROW_BLK, D), lambda i: (i, 0))
    w1sp = pl.BlockSpec((D, 2 * D), lambda i: (0, 0))
    b1sp = pl.BlockSpec((1, 2 * D), lambda i: (0, 0))
    w2sp = pl.BlockSpec((2 * D, D), lambda i: (0, 0))
    wsp = pl.BlockSpec((D, D), lambda i: (0, 0))
    bsp = pl.BlockSpec((1, D), lambda i: (0, 0))
    out = jax.ShapeDtypeStruct((N, D), jnp.float32)
    b = lambda name: p[name].reshape(1, -1)
    return pl.pallas_call(
        _fused_body,
        grid=(N_BLKS,),
        in_specs=[row, row, row, row, row, row,
                  bsp, w1sp, b1sp, w2sp, bsp, bsp, bsp, bsp, bsp,
                  wsp, bsp, wsp, bsp, wsp, bsp,
                  wsp, bsp, wsp, bsp, wsp, bsp,
                  bsp, bsp, bsp, bsp],
        out_specs=[row, row],
        out_shape=[out, out],
    )(high_emb, low_emb, agg, num, rinv, skip,
      jnp.broadcast_to(1.0 + p['gin_eps'], (1, D)), p['gin_w1'], b('gin_b1'), p['gin_w2'], b('gin_b2'),
      b('gin_ln_g'), b('gin_ln_b'), b('tc_ln_g'), b('tc_ln_b'),
      p['c_h2l_qw'], b('c_h2l_qb'), p['c_h2l_kw'], b('c_h2l_kb'),
      p['c_h2l_vw'], b('c_h2l_vb'),
      p['c_l2h_qw'], b('c_l2h_qb'), p['c_l2h_kw'], b('c_l2h_kb'),
      p['c_l2h_vw'], b('c_l2h_vb'),
      b('nh_g'), b('nh_b'), b('nl_g'), b('nl_b'))


def kernel(high_emb, low_emb, spatial_edge_index, grn_edge_index, params):
    p = params
    src, dst = spatial_edge_index[0], spatial_edge_index[1]
    gs, gd = grn_edge_index[0], grn_edge_index[1]

    q, k, v, skip = _proj(low_emb, p)

    padn = EPAD - E
    src_p = jnp.pad(src, (0, padn))
    dst_p = jnp.pad(dst, (0, padn), constant_values=DUMP)
    gs_p = jnp.pad(gs, (0, padn))
    gdg_p = jnp.pad(gd, (0, padn))
    gd_p = jnp.pad(gd, (0, padn), constant_values=DUMP)
    z128 = jnp.zeros((NP, DH), jnp.float32)
    iota_e = jnp.arange(EPAD, dtype=jnp.int32)

    a0, a1 = _gin_agg(high_emb[:, :DH], high_emb[:, DH:], src_p, dst_p, z128)
    agg = jnp.concatenate([a0[:N], a1[:N]], axis=1)

    qe0, qe1, ke0, ke1, ve0, ve1 = _gather3(
        q[:, :DH], q[:, DH:], k[:, :DH], k[:, DH:], v[:, :DH], v[:, DH:],
        gdg_p, gs_p)
    qe = jnp.concatenate([qe0, qe1], axis=1)
    ke = jnp.concatenate([ke0, ke1], axis=1)
    ve = jnp.concatenate([ve0, ve1], axis=1)

    lane = jnp.arange(D, dtype=jnp.int32)
    mmat = (lane[:, None] // C == lane[None, :] // C).astype(jnp.float32)
    pmat = ((lane[:, None] % C == 0) &
            (lane[None, :] == lane[:, None] // C)).astype(jnp.float32)

    we, exd = _edge_tc(qe, ke, ve, mmat, pmat)

    n0, n1 = _gin_agg(we[:, :DH], we[:, DH:], iota_e, gd_p, z128)
    num = jnp.concatenate([n0[:N], n1[:N]], axis=1)
    d0, _ = _gin_agg(exd[:, :DH], exd[:, DH:], iota_e, gd_p, z128)
    denom = d0[:N, :H]

    rinv = jnp.repeat(1.0 / jnp.maximum(denom, 1e-16), C, axis=1)
    high_new, low_new = _fused(high_emb, low_emb, agg, num, rinv, skip, p)
    return high_new, low_new


# speedup vs baseline: 10.7588x; 1.1866x over previous
import functools
import math

import jax
import jax.numpy as jnp
from jax import lax
from jax.experimental import pallas as pl
from jax.experimental.pallas import tpu as pltpu
from jax.experimental.pallas import tpu_sc as plsc

N = 10000
D = 256
E = 160000
H = 4
C = D // H

ROW_BLK = 400
N_BLKS = N // ROW_BLK

NC = 2
NS = 16
DH = D // 2
NP = 10240
RPT = NP // NS
DUMP = 10100
EPAD = 161280
PT_SC = EPAD // NS
BSC = 80

EB = 640
E_BLKS = EPAD // EB

_MESH = plsc.VectorSubcoreMesh(core_axis_name="c", subcore_axis_name="s",
                               num_cores=NC, num_subcores=NS)


def _f32(shape):
    return jax.ShapeDtypeStruct(shape, jnp.float32)


def _gin_agg(h0, h1, src, dst, z128):
    @functools.partial(
        pl.kernel,
        out_type=[_f32((NP, DH)), _f32((NP, DH))],
        mesh=_MESH,
        compiler_params=pltpu.CompilerParams(needs_layout_passes=False),
        scratch_types=[
            pltpu.VMEM_SHARED((NP, DH), jnp.float32),
            pltpu.VMEM((BSC,), jnp.int32),
            pltpu.VMEM((BSC,), jnp.int32),
            pltpu.VMEM((BSC, DH), jnp.float32),
            pltpu.VMEM((BSC,), jnp.int32),
            pltpu.VMEM((BSC,), jnp.int32),
            pltpu.VMEM((BSC, DH), jnp.float32),
            pltpu.SemaphoreType.DMA,
            pltpu.SemaphoreType.DMA,
        ],
    )
    def body(h0_h, h1_h, src_h, dst_h, z_h, o0_h, o1_h,
             acc, idx_a, dst_a, rows_a, idx_b, dst_b, rows_b, sem_a, sem_b):
        c = lax.axis_index("c")
        s = lax.axis_index("s")
        pltpu.sync_copy(z_h.at[pl.ds(s * RPT, RPT)], acc.at[pl.ds(s * RPT, RPT)])
        plsc.subcore_barrier()
        for cid, tbl, out in ((0, h0_h, o0_h), (1, h1_h, o1_h)):
            @pl.when(c == cid)
            def _():
                def step(i, carry):
                    base = s * PT_SC + 2 * i * BSC
                    pltpu.sync_copy(src_h.at[pl.ds(base, BSC)], idx_a)
                    pltpu.sync_copy(dst_h.at[pl.ds(base, BSC)], dst_a)
                    ca = pltpu.async_copy(tbl.at[idx_a], rows_a, sem_a)
                    pltpu.sync_copy(src_h.at[pl.ds(base + BSC, BSC)], idx_b)
                    pltpu.sync_copy(dst_h.at[pl.ds(base + BSC, BSC)], dst_b)
                    cb = pltpu.async_copy(tbl.at[idx_b], rows_b, sem_b)
                    ca.wait()
                    pltpu.sync_copy(rows_a, acc.at[dst_a], add=True)
                    cb.wait()
                    pltpu.sync_copy(rows_b, acc.at[dst_b], add=True)
                    return carry
                lax.fori_loop(0, PT_SC // (2 * BSC), step, 0)
                plsc.subcore_barrier()
                pltpu.sync_copy(acc.at[pl.ds(s * RPT, RPT)],
                                out.at[pl.ds(s * RPT, RPT)])

    return body(h0, h1, src, dst, z128)


def _gather3(q0, q1, k0, k1, v0, v1, gd, gs):
    @functools.partial(
        pl.kernel,
        out_type=[_f32((EPAD, DH)) for _ in range(6)],
        mesh=_MESH,
        compiler_params=pltpu.CompilerParams(needs_layout_passes=False),
        scratch_types=[
            pltpu.VMEM((BSC,), jnp.int32),
            pltpu.VMEM((BSC,), jnp.int32),
            pltpu.VMEM((BSC, DH), jnp.float32),
            pltpu.VMEM((BSC, DH), jnp.float32),
            pltpu.VMEM((BSC, DH), jnp.float32),
            pltpu.SemaphoreType.DMA,
            pltpu.SemaphoreType.DMA,
            pltpu.SemaphoreType.DMA,
        ],
    )
    def body(q0_h, q1_h, k0_h, k1_h, v0_h, v1_h, gd_h, gs_h,
             oq0, oq1, ok0, ok1, ov0, ov1,
             gdv, gsv, rq, rk, rv, semq, semk, semv):
        c = lax.axis_index("c")
        s = lax.axis_index("s")
        for cid, tq, tk, tv, oq, okk, ov in (
                (0, q0_h, k0_h, v0_h, oq0, ok0, ov0),
                (1, q1_h, k1_h, v1_h, oq1, ok1, ov1)):
            @pl.when(c == cid)
            def _():
                def step(i, carry):
                    base = s * PT_SC + i * BSC
                    pltpu.sync_copy(gd_h.at[pl.ds(base, BSC)], gdv)
                    pltpu.sync_copy(gs_h.at[pl.ds(base, BSC)], gsv)
                    cq = pltpu.async_copy(tq.at[gdv], rq, semq)
                    ck = pltpu.async_copy(tk.at[gsv], rk, semk)
                    cv = pltpu.async_copy(tv.at[gsv], rv, semv)
                    cq.wait()
                    pltpu.sync_copy(rq, oq.at[pl.ds(base, BSC)])
                    ck.wait()
                    pltpu.sync_copy(rk, okk.at[pl.ds(base, BSC)])
                    cv.wait()
                    pltpu.sync_copy(rv, ov.at[pl.ds(base, BSC)])
                    return carry
                lax.fori_loop(0, PT_SC // BSC, step, 0)

    return body(q0, q1, k0, k1, v0, v1, gd, gs)


PT_DN = EPAD // (NC * NS)


def _den_agg(exd, dst, z128):
    @functools.partial(
        pl.kernel,
        out_type=[_f32((NP, DH)), _f32((NP, DH))],
        mesh=_MESH,
        compiler_params=pltpu.CompilerParams(needs_layout_passes=False),
        scratch_types=[
            pltpu.VMEM_SHARED((NP, DH), jnp.float32),
            pltpu.VMEM((BSC,), jnp.int32),
            pltpu.VMEM((BSC, DH), jnp.float32),
            pltpu.VMEM((BSC,), jnp.int32),
            pltpu.VMEM((BSC, DH), jnp.float32),
            pltpu.SemaphoreType.DMA,
            pltpu.SemaphoreType.DMA,
        ],
    )
    def body(exd_h, dst_h, z_h, d0_h, d1_h,
             acc, dst_a, rows_a, dst_b, rows_b, sem_a, sem_b):
        c = lax.axis_index("c")
        s = lax.axis_index("s")
        pltpu.sync_copy(z_h.at[pl.ds(s * RPT, RPT)], acc.at[pl.ds(s * RPT, RPT)])
        plsc.subcore_barrier()
        tile0 = c * (NS * PT_DN) + s * PT_DN

        def step(i, carry):
            base = tile0 + 2 * i * BSC
            pltpu.sync_copy(dst_h.at[pl.ds(base, BSC)], dst_a)
            ca = pltpu.async_copy(exd_h.at[pl.ds(base, BSC)], rows_a, sem_a)
            pltpu.sync_copy(dst_h.at[pl.ds(base + BSC, BSC)], dst_b)
            cb = pltpu.async_copy(exd_h.at[pl.ds(base + BSC, BSC)], rows_b, sem_b)
            ca.wait()
            pltpu.sync_copy(rows_a, acc.at[dst_a], add=True)
            cb.wait()
            pltpu.sync_copy(rows_b, acc.at[dst_b], add=True)
            return carry

        lax.fori_loop(0, PT_DN // (2 * BSC), step, 0)
        base = tile0 + (PT_DN // (2 * BSC)) * 2 * BSC
        pltpu.sync_copy(dst_h.at[pl.ds(base, BSC)], dst_a)
        pltpu.sync_copy(exd_h.at[pl.ds(base, BSC)], rows_a)
        pltpu.sync_copy(rows_a, acc.at[dst_a], add=True)
        plsc.subcore_barrier()
        for cid, out in ((0, d0_h), (1, d1_h)):
            @pl.when(c == cid)
            def _():
                pltpu.sync_copy(acc.at[pl.ds(s * RPT, RPT)],
                                out.at[pl.ds(s * RPT, RPT)])

    return body(exd, dst, z128)


def _edge_body(qe_ref, ke_ref, ve_ref, m_ref, p_ref, we_ref, exd_ref):
    prod = qe_ref[...] * ke_ref[...]
    s = jnp.dot(prod, m_ref[...], preferred_element_type=jnp.float32)
    exf = jnp.exp(s * (1.0 / math.sqrt(C)))
    we_ref[...] = exf * ve_ref[...]
    exd_ref[...] = jnp.dot(exf, p_ref[...], preferred_element_type=jnp.float32)


def _edge_tc(qe, ke, ve, mmat, pmat):
    row = pl.BlockSpec((EB, D), lambda i: (i, 0))
    rowh = pl.BlockSpec((EB, DH), lambda i: (i, 0))
    msp = pl.BlockSpec((D, D), lambda i: (0, 0))
    psp = pl.BlockSpec((D, DH), lambda i: (0, 0))
    out = jax.ShapeDtypeStruct((EPAD, D), jnp.float32)
    outh = jax.ShapeDtypeStruct((EPAD, DH), jnp.float32)
    return pl.pallas_call(
        _edge_body,
        grid=(E_BLKS,),
        in_specs=[row, row, row, msp, psp],
        out_specs=[row, rowh],
        out_shape=[out, outh],
    )(qe, ke, ve, mmat, pmat)


def _proj_body(low_ref, wq, bq, wk, bk, wv, bv, ws, bs, q_o, k_o, v_o, s_o):
    x = low_ref[...]
    q_o[...] = jnp.dot(x, wq[...], preferred_element_type=jnp.float32) + bq[...]
    k_o[...] = jnp.dot(x, wk[...], preferred_element_type=jnp.float32) + bk[...]
    v_o[...] = jnp.dot(x, wv[...], preferred_element_type=jnp.float32) + bv[...]
    s_o[...] = jnp.dot(x, ws[...], preferred_element_type=jnp.float32) + bs[...]


def _proj(low_emb, p):
    row = pl.BlockSpec((ROW_BLK, D), lambda i: (i, 0))
    wsp = pl.BlockSpec((D, D), lambda i: (0, 0))
    bsp = pl.BlockSpec((1, D), lambda i: (0, 0))
    out = jax.ShapeDtypeStruct((N, D), jnp.float32)
    return pl.pallas_call(
        _proj_body,
        grid=(N_BLKS,),
        in_specs=[row, wsp, bsp, wsp, bsp, wsp, bsp, wsp, bsp],
        out_specs=[row, row, row, row],
        out_shape=[out, out, out, out],
    )(low_emb,
      p['tc_wq'], p['tc_bq'].reshape(1, D),
      p['tc_wk'], p['tc_bk'].reshape(1, D),
      p['tc_wv'], p['tc_bv'].reshape(1, D),
      p['tc_wskip'], p['tc_bskip'].reshape(1, D))


def _ln(x, g, b):
    mu = jnp.mean(x, axis=-1, keepdims=True)
    var = jnp.mean((x - mu) ** 2, axis=-1, keepdims=True)
    return (x - mu) * jax.lax.rsqrt(var + 1e-5) * g + b


def _fused_body(high_ref, low_ref, agg_ref, num_ref, rinv_ref, skip_ref,
                eps_row, w1, b1, w2, b2, gln_g, gln_b, tln_g, tln_b,
                h2l_qw, h2l_qb, h2l_kw, h2l_kb, h2l_vw, h2l_vb,
                l2h_qw, l2h_qb, l2h_kw, l2h_kb, l2h_vw, l2h_vb,
                nh_g, nh_b, nl_g, nl_b,
                high_new_ref, low_new_ref):
    high = high_ref[...]
    low = low_ref[...]
    h = eps_row[...] * high + agg_ref[...]
    h = jnp.dot(h, w1[...], preferred_element_type=jnp.float32) + b1[...]
    h = 0.5 * h * (1.0 + lax.erf(h * (1.0 / math.sqrt(2.0))))
    h = jnp.dot(h, w2[...], preferred_element_type=jnp.float32) + b2[...]
    high_out = _ln(h, gln_g[...], gln_b[...])
    out = num_ref[...] * rinv_ref[...] + skip_ref[...]
    low_out = _ln(out, tln_g[...], tln_b[...])
    scale = D ** (-0.5)
    q_h2l = jnp.dot(high_out, h2l_qw[...], preferred_element_type=jnp.float32) + h2l_qb[...]
    k_h2l = jnp.dot(low_out, h2l_kw[...], preferred_element_type=jnp.float32) + h2l_kb[...]
    v_h2l = jnp.dot(low_out, h2l_vw[...], preferred_element_type=jnp.float32) + h2l_vb[...]
    a_h2l = jax.nn.sigmoid(jnp.sum(q_h2l * k_h2l, axis=-1, keepdims=True) * scale)
    high_cross = a_h2l * v_h2l
    q_l2h = jnp.dot(low_out, l2h_qw[...], preferred_element_type=jnp.float32) + l2h_qb[...]
    k_l2h = jnp.dot(high_out, l2h_kw[...], preferred_element_type=jnp.float32) + l2h_kb[...]
    v_l2h = jnp.dot(high_out, l2h_vw[...], preferred_element_type=jnp.float32) + l2h_vb[...]
    a_l2h = jax.nn.sigmoid(jnp.sum(q_l2h * k_l2h, axis=-1, keepdims=True) * scale)
    low_cross = a_l2h * v_l2h
    high_new_ref[...] = _ln(high + high_out + high_cross, nh_g[...], nh_b[...])
    low_new_ref[...] = _ln(low + low_out + low_cross, nl_g[...], nl_b[...])


def _fused(high_emb, low_emb, agg, num, rinv, skip, p):
    row = pl.BlockSpec((ROW_BLK, D), lambda i: (i, 0))
    w1sp = pl.BlockSpec((D, 2 * D), lambda i: (0, 0))
    b1sp = pl.BlockSpec((1, 2 * D), lambda i: (0, 0))
    w2sp = pl.BlockSpec((2 * D, D), lambda i: (0, 0))
    wsp = pl.BlockSpec((D, D), lambda i: (0, 0))
    bsp = pl.BlockSpec((1, D), lambda i: (0, 0))
    out = jax.ShapeDtypeStruct((N, D), jnp.float32)
    b = lambda name: p[name].reshape(1, -1)
    return pl.pallas_call(
        _fused_body,
        grid=(N_BLKS,),
        in_specs=[row, row, row, row, row, row,
                  bsp, w1sp, b1sp, w2sp, bsp, bsp, bsp, bsp, bsp,
                  wsp, bsp, wsp, bsp, wsp, bsp,
                  wsp, bsp, wsp, bsp, wsp, bsp,
                  bsp, bsp, bsp, bsp],
        out_specs=[row, row],
        out_shape=[out, out],
    )(high_emb, low_emb, agg, num, rinv, skip,
      jnp.broadcast_to(1.0 + p['gin_eps'], (1, D)), p['gin_w1'], b('gin_b1'), p['gin_w2'], b('gin_b2'),
      b('gin_ln_g'), b('gin_ln_b'), b('tc_ln_g'), b('tc_ln_b'),
      p['c_h2l_qw'], b('c_h2l_qb'), p['c_h2l_kw'], b('c_h2l_kb'),
      p['c_h2l_vw'], b('c_h2l_vb'),
      p['c_l2h_qw'], b('c_l2h_qb'), p['c_l2h_kw'], b('c_l2h_kb'),
      p['c_l2h_vw'], b('c_l2h_vb'),
      b('nh_g'), b('nh_b'), b('nl_g'), b('nl_b'))


def kernel(high_emb, low_emb, spatial_edge_index, grn_edge_index, params):
    p = params
    src, dst = spatial_edge_index[0], spatial_edge_index[1]
    gs, gd = grn_edge_index[0], grn_edge_index[1]

    q, k, v, skip = _proj(low_emb, p)

    padn = EPAD - E
    src_p = jnp.pad(src, (0, padn))
    dst_p = jnp.pad(dst, (0, padn), constant_values=DUMP)
    gs_p = jnp.pad(gs, (0, padn))
    gdg_p = jnp.pad(gd, (0, padn))
    gd_p = jnp.pad(gd, (0, padn), constant_values=DUMP)
    z128 = jnp.zeros((NP, DH), jnp.float32)

    a0, a1 = _gin_agg(high_emb[:, :DH], high_emb[:, DH:], src_p, dst_p, z128)
    agg = jnp.concatenate([a0[:N], a1[:N]], axis=1)

    qe0, qe1, ke0, ke1, ve0, ve1 = _gather3(
        q[:, :DH], q[:, DH:], k[:, :DH], k[:, DH:], v[:, :DH], v[:, DH:],
        gdg_p, gs_p)
    qe = jnp.concatenate([qe0, qe1], axis=1)
    ke = jnp.concatenate([ke0, ke1], axis=1)
    ve = jnp.concatenate([ve0, ve1], axis=1)

    lane = jnp.arange(D, dtype=jnp.int32)
    mmat = (lane[:, None] // C == lane[None, :] // C).astype(jnp.float32)
    pmat = ((lane[:, None] % C == 0) &
            (lane[None, :DH] == lane[:, None] // C)).astype(jnp.float32)

    we, exd = _edge_tc(qe, ke, ve, mmat, pmat)

    iota_e = jnp.arange(EPAD, dtype=jnp.int32)
    n0, n1 = _gin_agg(we[:, :DH], we[:, DH:], iota_e, gd_p, z128)
    num = jnp.concatenate([n0[:N], n1[:N]], axis=1)
    d0, d1 = _den_agg(exd, gd_p, z128)
    denom = (d0 + d1)[:N, :H]

    rinv = jnp.repeat(1.0 / jnp.maximum(denom, 1e-16), C, axis=1)
    high_new, low_new = _fused(high_emb, low_emb, agg, num, rinv, skip, p)
    return high_new, low_new


# speedup vs baseline: 13.8096x; 1.2836x over previous
import functools
import math

import jax
import jax.numpy as jnp
from jax import lax
from jax.experimental import pallas as pl
from jax.experimental.pallas import tpu as pltpu
from jax.experimental.pallas import tpu_sc as plsc

N = 10000
D = 256
E = 160000
H = 4
C = D // H

ROW_BLK = 400
N_BLKS = N // ROW_BLK

NC = 2
NS = 16
DH = D // 2
NP = 10240
RPT = NP // NS
DUMP = 10100
EPAD = 161280
PT_SC = EPAD // NS
BSC = 80

EB = 640
E_BLKS = EPAD // EB

_MESH = plsc.VectorSubcoreMesh(core_axis_name="c", subcore_axis_name="s",
                               num_cores=NC, num_subcores=NS)


def _f32(shape):
    return jax.ShapeDtypeStruct(shape, jnp.float32)


def _gin_agg(h0, h1, src, dst, z128):
    @functools.partial(
        pl.kernel,
        out_type=[_f32((NP, DH)), _f32((NP, DH))],
        mesh=_MESH,
        compiler_params=pltpu.CompilerParams(needs_layout_passes=False),
        scratch_types=[
            pltpu.VMEM_SHARED((NP, DH), jnp.float32),
            pltpu.VMEM((BSC,), jnp.int32),
            pltpu.VMEM((BSC,), jnp.int32),
            pltpu.VMEM((BSC, DH), jnp.float32),
            pltpu.VMEM((BSC,), jnp.int32),
            pltpu.VMEM((BSC,), jnp.int32),
            pltpu.VMEM((BSC, DH), jnp.float32),
            pltpu.SemaphoreType.DMA,
            pltpu.SemaphoreType.DMA,
        ],
    )
    def body(h0_h, h1_h, src_h, dst_h, z_h, o0_h, o1_h,
             acc, idx_a, dst_a, rows_a, idx_b, dst_b, rows_b, sem_a, sem_b):
        c = lax.axis_index("c")
        s = lax.axis_index("s")
        pltpu.sync_copy(z_h.at[pl.ds(s * RPT, RPT)], acc.at[pl.ds(s * RPT, RPT)])
        plsc.subcore_barrier()
        for cid, tbl, out in ((0, h0_h, o0_h), (1, h1_h, o1_h)):
            @pl.when(c == cid)
            def _():
                def step(i, carry):
                    base = s * PT_SC + 2 * i * BSC
                    pltpu.sync_copy(src_h.at[pl.ds(base, BSC)], idx_a)
                    pltpu.sync_copy(dst_h.at[pl.ds(base, BSC)], dst_a)
                    ca = pltpu.async_copy(tbl.at[idx_a], rows_a, sem_a)
                    pltpu.sync_copy(src_h.at[pl.ds(base + BSC, BSC)], idx_b)
                    pltpu.sync_copy(dst_h.at[pl.ds(base + BSC, BSC)], dst_b)
                    cb = pltpu.async_copy(tbl.at[idx_b], rows_b, sem_b)
                    ca.wait()
                    pltpu.sync_copy(rows_a, acc.at[dst_a], add=True)
                    cb.wait()
                    pltpu.sync_copy(rows_b, acc.at[dst_b], add=True)
                    return carry
                lax.fori_loop(0, PT_SC // (2 * BSC), step, 0)
                plsc.subcore_barrier()
                pltpu.sync_copy(acc.at[pl.ds(s * RPT, RPT)],
                                out.at[pl.ds(s * RPT, RPT)])

    return body(h0, h1, src, dst, z128)


def _gather3(q0, q1, k, v, gd, gs):
    @functools.partial(
        pl.kernel,
        out_type=[_f32((EPAD, DH)), _f32((EPAD, DH)),
                  _f32((EPAD, D)), _f32((EPAD, D))],
        mesh=_MESH,
        compiler_params=pltpu.CompilerParams(needs_layout_passes=False),
        scratch_types=[
            pltpu.VMEM((BSC,), jnp.int32),
            pltpu.VMEM((BSC,), jnp.int32),
            pltpu.VMEM((BSC, DH), jnp.float32),
            pltpu.VMEM((BSC, D), jnp.float32),
            pltpu.VMEM((BSC,), jnp.int32),
            pltpu.VMEM((BSC,), jnp.int32),
            pltpu.VMEM((BSC, DH), jnp.float32),
            pltpu.VMEM((BSC, D), jnp.float32),
            pltpu.SemaphoreType.DMA,
            pltpu.SemaphoreType.DMA,
            pltpu.SemaphoreType.DMA,
            pltpu.SemaphoreType.DMA,
        ],
    )
    def body(q0_h, q1_h, k_h, v_h, gd_h, gs_h,
             oq0, oq1, oke, ove,
             gdv_a, gsv_a, rq_a, rw_a, gdv_b, gsv_b, rq_b, rw_b,
             semq_a, semw_a, semq_b, semw_b):
        c = lax.axis_index("c")
        s = lax.axis_index("s")
        for cid, tq, tw, oq, ow in ((0, q0_h, k_h, oq0, oke),
                                    (1, q1_h, v_h, oq1, ove)):
            @pl.when(c == cid)
            def _():
                def step(i, carry):
                    base = s * PT_SC + 2 * i * BSC
                    pltpu.sync_copy(gd_h.at[pl.ds(base, BSC)], gdv_a)
                    pltpu.sync_copy(gs_h.at[pl.ds(base, BSC)], gsv_a)
                    cqa = pltpu.async_copy(tq.at[gdv_a], rq_a, semq_a)
                    cwa = pltpu.async_copy(tw.at[gsv_a], rw_a, semw_a)
                    pltpu.sync_copy(gd_h.at[pl.ds(base + BSC, BSC)], gdv_b)
                    pltpu.sync_copy(gs_h.at[pl.ds(base + BSC, BSC)], gsv_b)
                    cqb = pltpu.async_copy(tq.at[gdv_b], rq_b, semq_b)
                    cwb = pltpu.async_copy(tw.at[gsv_b], rw_b, semw_b)
                    cqa.wait()
                    pltpu.sync_copy(rq_a, oq.at[pl.ds(base, BSC)])
                    cwa.wait()
                    pltpu.sync_copy(rw_a, ow.at[pl.ds(base, BSC)])
                    cqb.wait()
                    pltpu.sync_copy(rq_b, oq.at[pl.ds(base + BSC, BSC)])
                    cwb.wait()
                    pltpu.sync_copy(rw_b, ow.at[pl.ds(base + BSC, BSC)])
                    return carry
                lax.fori_loop(0, PT_SC // (2 * BSC), step, 0)

    return body(q0, q1, k, v, gd, gs)


PT_DN = EPAD // (NC * NS)


def _den_agg(exd, dst, z128):
    @functools.partial(
        pl.kernel,
        out_type=[_f32((NP, DH)), _f32((NP, DH))],
        mesh=_MESH,
        compiler_params=pltpu.CompilerParams(needs_layout_passes=False),
        scratch_types=[
            pltpu.VMEM_SHARED((NP, DH), jnp.float32),
            pltpu.VMEM((BSC,), jnp.int32),
            pltpu.VMEM((BSC, DH), jnp.float32),
            pltpu.VMEM((BSC,), jnp.int32),
            pltpu.VMEM((BSC, DH), jnp.float32),
            pltpu.SemaphoreType.DMA,
            pltpu.SemaphoreType.DMA,
        ],
    )
    def body(exd_h, dst_h, z_h, d0_h, d1_h,
             acc, dst_a, rows_a, dst_b, rows_b, sem_a, sem_b):
        c = lax.axis_index("c")
        s = lax.axis_index("s")
        pltpu.sync_copy(z_h.at[pl.ds(s * RPT, RPT)], acc.at[pl.ds(s * RPT, RPT)])
        plsc.subcore_barrier()
        tile0 = c * (NS * PT_DN) + s * PT_DN

        def step(i, carry):
            base = tile0 + 2 * i * BSC
            pltpu.sync_copy(dst_h.at[pl.ds(base, BSC)], dst_a)
            ca = pltpu.async_copy(exd_h.at[pl.ds(base, BSC)], rows_a, sem_a)
            pltpu.sync_copy(dst_h.at[pl.ds(base + BSC, BSC)], dst_b)
            cb = pltpu.async_copy(exd_h.at[pl.ds(base + BSC, BSC)], rows_b, sem_b)
            ca.wait()
            pltpu.sync_copy(rows_a, acc.at[dst_a], add=True)
            cb.wait()
            pltpu.sync_copy(rows_b, acc.at[dst_b], add=True)
            return carry

        lax.fori_loop(0, PT_DN // (2 * BSC), step, 0)
        base = tile0 + (PT_DN // (2 * BSC)) * 2 * BSC
        pltpu.sync_copy(dst_h.at[pl.ds(base, BSC)], dst_a)
        pltpu.sync_copy(exd_h.at[pl.ds(base, BSC)], rows_a)
        pltpu.sync_copy(rows_a, acc.at[dst_a], add=True)
        plsc.subcore_barrier()
        for cid, out in ((0, d0_h), (1, d1_h)):
            @pl.when(c == cid)
            def _():
                pltpu.sync_copy(acc.at[pl.ds(s * RPT, RPT)],
                                out.at[pl.ds(s * RPT, RPT)])

    return body(exd, dst, z128)


def _edge_body(qe_ref, ke_ref, ve_ref, m_ref, p_ref, we_ref, exd_ref):
    prod = qe_ref[...] * ke_ref[...]
    s = jnp.dot(prod, m_ref[...], preferred_element_type=jnp.float32)
    exf = jnp.exp(s * (1.0 / math.sqrt(C)))
    we_ref[...] = exf * ve_ref[...]
    exd_ref[...] = jnp.dot(exf, p_ref[...], preferred_element_type=jnp.float32)


def _edge_tc(qe, ke, ve, mmat, pmat):
    row = pl.BlockSpec((EB, D), lambda i: (i, 0))
    rowh = pl.BlockSpec((EB, DH), lambda i: (i, 0))
    msp = pl.BlockSpec((D, D), lambda i: (0, 0))
    psp = pl.BlockSpec((D, DH), lambda i: (0, 0))
    out = jax.ShapeDtypeStruct((EPAD, D), jnp.float32)
    outh = jax.ShapeDtypeStruct((EPAD, DH), jnp.float32)
    return pl.pallas_call(
        _edge_body,
        grid=(E_BLKS,),
        in_specs=[row, row, row, msp, psp],
        out_specs=[row, rowh],
        out_shape=[out, outh],
    )(qe, ke, ve, mmat, pmat)


def _proj_body(low_ref, wq, bq, wk, bk, wv, bv, ws, bs, q_o, k_o, v_o, s_o):
    x = low_ref[...]
    q_o[...] = jnp.dot(x, wq[...], preferred_element_type=jnp.float32) + bq[...]
    k_o[...] = jnp.dot(x, wk[...], preferred_element_type=jnp.float32) + bk[...]
    v_o[...] = jnp.dot(x, wv[...], preferred_element_type=jnp.float32) + bv[...]
    s_o[...] = jnp.dot(x, ws[...], preferred_element_type=jnp.float32) + bs[...]


def _proj(low_emb, p):
    row = pl.BlockSpec((ROW_BLK, D), lambda i: (i, 0))
    wsp = pl.BlockSpec((D, D), lambda i: (0, 0))
    bsp = pl.BlockSpec((1, D), lambda i: (0, 0))
    out = jax.ShapeDtypeStruct((N, D), jnp.float32)
    return pl.pallas_call(
        _proj_body,
        grid=(N_BLKS,),
        in_specs=[row, wsp, bsp, wsp, bsp, wsp, bsp, wsp, bsp],
        out_specs=[row, row, row, row],
        out_shape=[out, out, out, out],
    )(low_emb,
      p['tc_wq'], p['tc_bq'].reshape(1, D),
      p['tc_wk'], p['tc_bk'].reshape(1, D),
      p['tc_wv'], p['tc_bv'].reshape(1, D),
      p['tc_wskip'], p['tc_bskip'].reshape(1, D))


def _ln(x, g, b):
    mu = jnp.mean(x, axis=-1, keepdims=True)
    var = jnp.mean((x - mu) ** 2, axis=-1, keepdims=True)
    return (x - mu) * jax.lax.rsqrt(var + 1e-5) * g + b


def _fused_body(high_ref, low_ref, agg_ref, num_ref, rinv_ref, skip_ref,
                eps_row, w1, b1, w2, b2, gln_g, gln_b, tln_g, tln_b,
                h2l_qw, h2l_qb, h2l_kw, h2l_kb, h2l_vw, h2l_vb,
                l2h_qw, l2h_qb, l2h_kw, l2h_kb, l2h_vw, l2h_vb,
                nh_g, nh_b, nl_g, nl_b,
                high_new_ref, low_new_ref):
    high = high_ref[...]
    low = low_ref[...]
    h = eps_row[...] * high + agg_ref[...]
    h = jnp.dot(h, w1[...], preferred_element_type=jnp.float32) + b1[...]
    h = 0.5 * h * (1.0 + lax.erf(h * (1.0 / math.sqrt(2.0))))
    h = jnp.dot(h, w2[...], preferred_element_type=jnp.float32) + b2[...]
    high_out = _ln(h, gln_g[...], gln_b[...])
    out = num_ref[...] * rinv_ref[...] + skip_ref[...]
    low_out = _ln(out, tln_g[...], tln_b[...])
    scale = D ** (-0.5)
    q_h2l = jnp.dot(high_out, h2l_qw[...], preferred_element_type=jnp.float32) + h2l_qb[...]
    k_h2l = jnp.dot(low_out, h2l_kw[...], preferred_element_type=jnp.float32) + h2l_kb[...]
    v_h2l = jnp.dot(low_out, h2l_vw[...], preferred_element_type=jnp.float32) + h2l_vb[...]
    a_h2l = jax.nn.sigmoid(jnp.sum(q_h2l * k_h2l, axis=-1, keepdims=True) * scale)
    high_cross = a_h2l * v_h2l
    q_l2h = jnp.dot(low_out, l2h_qw[...], preferred_element_type=jnp.float32) + l2h_qb[...]
    k_l2h = jnp.dot(high_out, l2h_kw[...], preferred_element_type=jnp.float32) + l2h_kb[...]
    v_l2h = jnp.dot(high_out, l2h_vw[...], preferred_element_type=jnp.float32) + l2h_vb[...]
    a_l2h = jax.nn.sigmoid(jnp.sum(q_l2h * k_l2h, axis=-1, keepdims=True) * scale)
    low_cross = a_l2h * v_l2h
    high_new_ref[...] = _ln(high + high_out + high_cross, nh_g[...], nh_b[...])
    low_new_ref[...] = _ln(low + low_out + low_cross, nl_g[...], nl_b[...])


def _fused(high_emb, low_emb, agg, num, rinv, skip, p):
    row = pl.BlockSpec((ROW_BLK, D), lambda i: (i, 0))
    w1sp = pl.BlockSpec((D, 2 * D), lambda i: (0, 0))
    b1sp = pl.BlockSpec((1, 2 * D), lambda i: (0, 0))
    w2sp = pl.BlockSpec((2 * D, D), lambda i: (0, 0))
    wsp = pl.BlockSpec((D, D), lambda i: (0, 0))
    bsp = pl.BlockSpec((1, D), lambda i: (0, 0))
    out = jax.ShapeDtypeStruct((N, D), jnp.float32)
    b = lambda name: p[name].reshape(1, -1)
    return pl.pallas_call(
        _fused_body,
        grid=(N_BLKS,),
        in_specs=[row, row, row, row, row, row,
                  bsp, w1sp, b1sp, w2sp, bsp, bsp, bsp, bsp, bsp,
                  wsp, bsp, wsp, bsp, wsp, bsp,
                  wsp, bsp, wsp, bsp, wsp, bsp,
                  bsp, bsp, bsp, bsp],
        out_specs=[row, row],
        out_shape=[out, out],
    )(high_emb, low_emb, agg, num, rinv, skip,
      jnp.broadcast_to(1.0 + p['gin_eps'], (1, D)), p['gin_w1'], b('gin_b1'), p['gin_w2'], b('gin_b2'),
      b('gin_ln_g'), b('gin_ln_b'), b('tc_ln_g'), b('tc_ln_b'),
      p['c_h2l_qw'], b('c_h2l_qb'), p['c_h2l_kw'], b('c_h2l_kb'),
      p['c_h2l_vw'], b('c_h2l_vb'),
      p['c_l2h_qw'], b('c_l2h_qb'), p['c_l2h_kw'], b('c_l2h_kb'),
      p['c_l2h_vw'], b('c_l2h_vb'),
      b('nh_g'), b('nh_b'), b('nl_g'), b('nl_b'))


def kernel(high_emb, low_emb, spatial_edge_index, grn_edge_index, params):
    p = params
    src, dst = spatial_edge_index[0], spatial_edge_index[1]
    gs, gd = grn_edge_index[0], grn_edge_index[1]

    q, k, v, skip = _proj(low_emb, p)

    padn = EPAD - E
    src_p = jnp.pad(src, (0, padn))
    dst_p = jnp.pad(dst, (0, padn), constant_values=DUMP)
    gs_p = jnp.pad(gs, (0, padn))
    gdg_p = jnp.pad(gd, (0, padn))
    gd_p = jnp.pad(gd, (0, padn), constant_values=DUMP)
    z128 = jnp.zeros((NP, DH), jnp.float32)

    a0, a1 = _gin_agg(high_emb[:, :DH], high_emb[:, DH:], src_p, dst_p, z128)
    agg = jnp.concatenate([a0[:N], a1[:N]], axis=1)

    qe0, qe1, ke, ve = _gather3(q[:, :DH], q[:, DH:], k, v, gdg_p, gs_p)
    qe = jnp.concatenate([qe0, qe1], axis=1)

    lane = jnp.arange(D, dtype=jnp.int32)
    mmat = (lane[:, None] // C == lane[None, :] // C).astype(jnp.float32)
    pmat = ((lane[:, None] % C == 0) &
            (lane[None, :DH] == lane[:, None] // C)).astype(jnp.float32)

    we, exd = _edge_tc(qe, ke, ve, mmat, pmat)

    iota_e = jnp.arange(EPAD, dtype=jnp.int32)
    n0, n1 = _gin_agg(we[:, :DH], we[:, DH:], iota_e, gd_p, z128)
    num = jnp.concatenate([n0[:N], n1[:N]], axis=1)
    d0, d1 = _den_agg(exd, gd_p, z128)
    denom = (d0 + d1)[:N, :H]

    rinv = jnp.repeat(1.0 / jnp.maximum(denom, 1e-16), C, axis=1)
    high_new, low_new = _fused(high_emb, low_emb, agg, num, rinv, skip, p)
    return high_new, low_new
